# baseline (speedup 1.0000x reference)
import numpy as np
import jax
import jax.numpy as jnp
from jax.experimental import pallas as pl
from jax.experimental.pallas import tpu as pltpu

LANES = 128
TB = 512
OP_DT = jnp.bfloat16


def _conv1_tabs():
    e1 = np.zeros((128, 256), np.float32)
    for p in range(2):
        for o in range(6):
            for jp in range(12):
                e1[o, p * 128 + o * 12 + jp] = 1.0
    c1 = np.zeros((5, 140, 25), np.float32)
    m1 = np.zeros((5, 140, 256), np.float32)
    for dj in range(5):
        for di in range(5):
            for jj in range(28):
                c1[dj, di * 28 + jj, di * 5 + dj] = 1.0
        for p in range(2):
            for o in range(6):
                for jp in range(12):
                    col = p * 128 + o * 12 + jp
                    jj = 2 * jp + p + dj
                    for di in range(5):
                        m1[dj, di * 28 + jj, col] = 1.0
    return e1, c1, m1


def _conv2_tabs():
    e2 = np.zeros((128, 256), np.float32)
    for p in range(2):
        for o2 in range(12):
            for j2p in range(4):
                e2[o2, p * 128 + o2 * 4 + j2p] = 1.0
    c2 = np.zeros((5, 640, 150), np.float32)
    m2 = np.zeros((5, 640, 256), np.float32)
    for dj in range(5):
        for di in range(5):
            for c in range(6):
                for ww in range(12):
                    c2[dj, di * 128 + c * 12 + ww, (di * 5 + dj) * 6 + c] = 1.0
        for p in range(2):
            for o2 in range(12):
                for j2p in range(4):
                    col = p * 128 + o2 * 4 + j2p
                    ww = 2 * j2p + p + dj
                    for di in range(5):
                        for c in range(6):
                            m2[dj, di * 128 + c * 12 + ww, col] = 1.0
    return e2, c2, m2


def _fc1_tab():
    pf = np.zeros((512, 192), np.float32)
    for h in range(4):
        for c in range(12):
            for w in range(4):
                pf[h * 128 + c * 4 + w, h * 48 + w * 12 + c] = 1.0
    return pf


_E1, _C1, _M1 = _conv1_tabs()
_E2, _C2, _M2 = _conv2_tabs()
_PF = _fc1_tab()


def _prep_kernel(c1w, c1b, c2w, c2b, fc1w, fc2w, outw, e1, c1t, m1, e2, c2t,
                 m2, pf, w1_o, b1_o, w2_o, b2_o, wf_o, wg_o, wo_o):
    f32 = jnp.float32
    v1 = jnp.dot(c1w[...], e1[...], preferred_element_type=f32)
    w1 = jnp.zeros((140, 256), f32)
    for dj in range(5):
        w1 = w1 + jnp.dot(c1t[dj], v1, preferred_element_type=f32) * m1[dj]
    w1_o[...] = w1.astype(w1_o.dtype)
    b1_o[...] = jnp.dot(c1b[...], e1[...],
                        preferred_element_type=f32)[:, :128]

    v2 = jnp.dot(c2w[...], e2[...], preferred_element_type=f32)
    w2 = jnp.zeros((640, 256), f32)
    for dj in range(5):
        w2 = w2 + jnp.dot(c2t[dj], v2, preferred_element_type=f32) * m2[dj]
    w2_o[...] = w2.astype(w2_o.dtype)
    b2_o[...] = jnp.dot(c2b[...], e2[...],
                        preferred_element_type=f32)[:, :128]

    wf_o[...] = jnp.dot(pf[...], fc1w[...],
                        preferred_element_type=f32).astype(wf_o.dtype)
    wg_o[...] = fc2w[...].astype(wg_o.dtype)
    wo_o[...] = outw[...].astype(wo_o.dtype)


def _full(shape):
    return pl.BlockSpec(shape, lambda: (0,) * len(shape))


def _prep(c1_w, c1_b, c2_w, c2_b, fc1_w, fc2_w, out_w):
    outs = pl.pallas_call(
        _prep_kernel,
        out_shape=[jax.ShapeDtypeStruct((140, 256), OP_DT),
                   jax.ShapeDtypeStruct((1, 128), jnp.float32),
                   jax.ShapeDtypeStruct((640, 256), OP_DT),
                   jax.ShapeDtypeStruct((1, 128), jnp.float32),
                   jax.ShapeDtypeStruct((512, 128), OP_DT),
                   jax.ShapeDtypeStruct((128, 128), OP_DT),
                   jax.ShapeDtypeStruct((128, 128), OP_DT)],
        in_specs=[_full((25, 128)), _full((1, 128)),
                  _full((150, 128)), _full((1, 128)),
                  _full((192, 128)), _full((128, 128)), _full((128, 128)),
                  _full((128, 256)), _full((5, 140, 25)), _full((5, 140, 256)),
                  _full((128, 256)), _full((5, 640, 150)), _full((5, 640, 256)),
                  _full((512, 192))],
        out_specs=[_full((140, 256)), _full((1, 128)),
                   _full((640, 256)), _full((1, 128)),
                   _full((512, 128)), _full((128, 128)), _full((128, 128))],
    )(c1_w, c1_b, c2_w, c2_b, fc1_w, fc2_w, out_w,
      _E1, _C1, _M1, _E2, _C2, _M2, _PF)
    return outs


def _lenet_kernel(x_ref, w1_ref, b1_ref, w2_ref, b2_ref, wf_ref, bf_ref,
                  wg_ref, bg_ref, wo_ref, bo_ref, o_ref, s1, s2):
    tb = x_ref.shape[0]
    xb = x_ref[...].astype(OP_DT).reshape(tb, 784)
    w1 = w1_ref[...]
    for r in range(12):
        m = None
        for i in (2 * r, 2 * r + 1):
            d = jnp.dot(xb[:, i * 28:i * 28 + 140], w1,
                        preferred_element_type=jnp.float32)
            mm = jnp.maximum(d[:, :LANES], d[:, LANES:])
            m = mm if m is None else jnp.maximum(m, mm)
        s1[:, r * LANES:(r + 1) * LANES] = (
            jnp.maximum(m + b1_ref[...], 0.0).astype(OP_DT))

    w2 = w2_ref[...]
    for r in range(4):
        m = None
        for i in (2 * r, 2 * r + 1):
            d = jnp.dot(s1[:, i * LANES:i * LANES + 640], w2,
                        preferred_element_type=jnp.float32)
            mm = jnp.maximum(d[:, :LANES], d[:, LANES:])
            m = mm if m is None else jnp.maximum(m, mm)
        s2[:, r * LANES:(r + 1) * LANES] = (
            jnp.maximum(m + b2_ref[...], 0.0).astype(OP_DT))

    h = jnp.dot(s2[...], wf_ref[...], preferred_element_type=jnp.float32)
    h = jnp.maximum(h + bf_ref[...], 0.0).astype(OP_DT)
    h = jnp.dot(h, wg_ref[...], preferred_element_type=jnp.float32)
    h = jnp.maximum(h + bg_ref[...], 0.0).astype(OP_DT)
    o = jnp.dot(h, wo_ref[...], preferred_element_type=jnp.float32)
    o_ref[...] = (o + bo_ref[...])[:, :10]


def kernel(c1_w, c1_b, c2_w, c2_b, fc1_w, fc1_b, fc2_w, fc2_b, out_w, out_b, x):
    B = x.shape[0]
    tb = TB if B % TB == 0 else B

    w1, b1, w2, b2, wf, wg, wo = _prep(c1_w, c1_b, c2_w, c2_b,
                                       fc1_w, fc2_w, out_w)

    return pl.pallas_call(
        _lenet_kernel,
        out_shape=jax.ShapeDtypeStruct((B, 10), jnp.float32),
        grid=(B // tb,),
        in_specs=[pl.BlockSpec((tb, 1, 28, 28), lambda i: (i, 0, 0, 0)),
                  pl.BlockSpec(w1.shape, lambda i: (0, 0)),
                  pl.BlockSpec(b1.shape, lambda i: (0, 0)),
                  pl.BlockSpec(w2.shape, lambda i: (0, 0)),
                  pl.BlockSpec(b2.shape, lambda i: (0, 0)),
                  pl.BlockSpec(wf.shape, lambda i: (0, 0)),
                  pl.BlockSpec(fc1_b.shape, lambda i: (0, 0)),
                  pl.BlockSpec(wg.shape, lambda i: (0, 0)),
                  pl.BlockSpec(fc2_b.shape, lambda i: (0, 0)),
                  pl.BlockSpec(wo.shape, lambda i: (0, 0)),
                  pl.BlockSpec(out_b.shape, lambda i: (0, 0))],
        out_specs=pl.BlockSpec((tb, 10), lambda i: (i, 0)),
        scratch_shapes=[pltpu.VMEM((tb, 12 * LANES), OP_DT),
                        pltpu.VMEM((tb, 4 * LANES), OP_DT)],
        compiler_params=pltpu.CompilerParams(
            dimension_semantics=("parallel",)),
    )(x, w1, b1, w2, b2, wf, fc1_b, wg, fc2_b, wo, out_b)


# speedup vs baseline: 1.6046x; 1.6046x over previous
import numpy as np
import jax
import jax.numpy as jnp
from jax.experimental import pallas as pl
from jax.experimental.pallas import tpu as pltpu

LANES = 128
TB = 512
OP_DT = jnp.bfloat16


def _conv1_tabs():
    e1 = np.zeros((128, 256), np.float32)
    for p in range(2):
        for o in range(6):
            for jp in range(12):
                e1[o, p * 128 + o * 12 + jp] = 1.0
    c1 = np.zeros((5, 140, 25), np.float32)
    m1 = np.zeros((5, 140, 256), np.float32)
    for dj in range(5):
        for di in range(5):
            for jj in range(28):
                c1[dj, di * 28 + jj, di * 5 + dj] = 1.0
        for p in range(2):
            for o in range(6):
                for jp in range(12):
                    col = p * 128 + o * 12 + jp
                    jj = 2 * jp + p + dj
                    for di in range(5):
                        m1[dj, di * 28 + jj, col] = 1.0
    return e1, c1, m1


def _conv2_tabs():
    e2 = np.zeros((128, 256), np.float32)
    for p in range(2):
        for o2 in range(12):
            for j2p in range(4):
                e2[o2, p * 128 + o2 * 4 + j2p] = 1.0
    c2 = np.zeros((5, 640, 150), np.float32)
    m2 = np.zeros((5, 640, 256), np.float32)
    for dj in range(5):
        for di in range(5):
            for c in range(6):
                for ww in range(12):
                    c2[dj, di * 128 + c * 12 + ww, (di * 5 + dj) * 6 + c] = 1.0
        for p in range(2):
            for o2 in range(12):
                for j2p in range(4):
                    col = p * 128 + o2 * 4 + j2p
                    ww = 2 * j2p + p + dj
                    for di in range(5):
                        for c in range(6):
                            m2[dj, di * 128 + c * 12 + ww, col] = 1.0
    return e2, c2, m2


def _fc1_tab():
    pf = np.zeros((512, 192), np.float32)
    for h in range(4):
        for c in range(12):
            for w in range(4):
                pf[h * 128 + c * 4 + w, h * 48 + w * 12 + c] = 1.0
    return pf


_E1, _C1, _M1 = _conv1_tabs()
_E2, _C2, _M2 = _conv2_tabs()
_PF = _fc1_tab()


def _prep_kernel(c1w, c1b, c2w, c2b, fc1w, fc2w, outw, e1, c1t, m1, e2, c2t,
                 m2, pf, w1_o, b1_o, w2_o, b2_o, wf_o, wg_o, wo_o):
    f32 = jnp.float32
    v1 = jnp.dot(c1w[...], e1[...], preferred_element_type=f32)
    w1 = jnp.zeros((140, 256), f32)
    for dj in range(5):
        w1 = w1 + jnp.dot(c1t[dj], v1, preferred_element_type=f32) * m1[dj]
    w1_o[...] = w1.astype(w1_o.dtype)
    b1_o[...] = jnp.dot(c1b[...], e1[...],
                        preferred_element_type=f32)[:, :128]

    v2 = jnp.dot(c2w[...], e2[...], preferred_element_type=f32)
    w2 = jnp.zeros((640, 256), f32)
    for dj in range(5):
        w2 = w2 + jnp.dot(c2t[dj], v2, preferred_element_type=f32) * m2[dj]
    w2_o[...] = w2.astype(w2_o.dtype)
    b2_o[...] = jnp.dot(c2b[...], e2[...],
                        preferred_element_type=f32)[:, :128]

    wf_o[...] = jnp.dot(pf[...], fc1w[...],
                        preferred_element_type=f32).astype(wf_o.dtype)
    wg_o[...] = fc2w[...].astype(wg_o.dtype)
    wo_o[...] = outw[...].astype(wo_o.dtype)


def _full(shape):
    return pl.BlockSpec(shape, lambda: (0,) * len(shape))


def _prep(c1_w, c1_b, c2_w, c2_b, fc1_w, fc2_w, out_w):
    outs = pl.pallas_call(
        _prep_kernel,
        out_shape=[jax.ShapeDtypeStruct((140, 256), OP_DT),
                   jax.ShapeDtypeStruct((1, 128), jnp.float32),
                   jax.ShapeDtypeStruct((640, 256), OP_DT),
                   jax.ShapeDtypeStruct((1, 128), jnp.float32),
                   jax.ShapeDtypeStruct((512, 128), OP_DT),
                   jax.ShapeDtypeStruct((128, 128), OP_DT),
                   jax.ShapeDtypeStruct((128, 128), OP_DT)],
        in_specs=[_full((25, 128)), _full((1, 128)),
                  _full((150, 128)), _full((1, 128)),
                  _full((192, 128)), _full((128, 128)), _full((128, 128)),
                  _full((128, 256)), _full((5, 140, 25)), _full((5, 140, 256)),
                  _full((128, 256)), _full((5, 640, 150)), _full((5, 640, 256)),
                  _full((512, 192))],
        out_specs=[_full((140, 256)), _full((1, 128)),
                   _full((640, 256)), _full((1, 128)),
                   _full((512, 128)), _full((128, 128)), _full((128, 128))],
    )(c1_w, c1_b, c2_w, c2_b, fc1_w, fc2_w, out_w,
      _E1, _C1, _M1, _E2, _C2, _M2, _PF)
    return outs


def _lenet_kernel(x_ref, w1_ref, b1_ref, w2_ref, b2_ref, wf_ref, bf_ref,
                  wg_ref, bg_ref, wo_ref, bo_ref, o_ref, s1, s2):
    tb = x_ref.shape[0]
    xb = x_ref[...].astype(OP_DT).reshape(tb, 784)
    w1 = w1_ref[...]
    for r in range(12):
        m = None
        for i in (2 * r, 2 * r + 1):
            d = jnp.dot(xb[:, i * 28:i * 28 + 140], w1,
                        preferred_element_type=jnp.float32)
            mm = jnp.maximum(d[:, :LANES], d[:, LANES:])
            m = mm if m is None else jnp.maximum(m, mm)
        s1[:, r * LANES:(r + 1) * LANES] = (
            jnp.maximum(m + b1_ref[...], 0.0).astype(OP_DT))

    w2 = w2_ref[...]
    for r in range(4):
        m = None
        for i in (2 * r, 2 * r + 1):
            d = jnp.dot(s1[:, i * LANES:i * LANES + 640], w2,
                        preferred_element_type=jnp.float32)
            mm = jnp.maximum(d[:, :LANES], d[:, LANES:])
            m = mm if m is None else jnp.maximum(m, mm)
        s2[:, r * LANES:(r + 1) * LANES] = (
            jnp.maximum(m + b2_ref[...], 0.0).astype(OP_DT))

    h = jnp.dot(s2[...], wf_ref[...], preferred_element_type=jnp.float32)
    h = jnp.maximum(h + bf_ref[...], 0.0).astype(OP_DT)
    h = jnp.dot(h, wg_ref[...], preferred_element_type=jnp.float32)
    h = jnp.maximum(h + bg_ref[...], 0.0).astype(OP_DT)
    o = jnp.dot(h, wo_ref[...], preferred_element_type=jnp.float32)
    o_ref[...] = (o + bo_ref[...])[:, :10]


def kernel(c1_w, c1_b, c2_w, c2_b, fc1_w, fc1_b, fc2_w, fc2_b, out_w, out_b, x):
    B = x.shape[0]
    tb = TB if B % TB == 0 else B
    x3 = x.reshape(B, 28, 28)

    w1, b1, w2, b2, wf, wg, wo = _prep(c1_w, c1_b, c2_w, c2_b,
                                       fc1_w, fc2_w, out_w)

    return pl.pallas_call(
        _lenet_kernel,
        out_shape=jax.ShapeDtypeStruct((B, 10), jnp.float32),
        grid=(B // tb,),
        in_specs=[pl.BlockSpec((tb, 28, 28), lambda i: (i, 0, 0)),
                  pl.BlockSpec(w1.shape, lambda i: (0, 0)),
                  pl.BlockSpec(b1.shape, lambda i: (0, 0)),
                  pl.BlockSpec(w2.shape, lambda i: (0, 0)),
                  pl.BlockSpec(b2.shape, lambda i: (0, 0)),
                  pl.BlockSpec(wf.shape, lambda i: (0, 0)),
                  pl.BlockSpec(fc1_b.shape, lambda i: (0, 0)),
                  pl.BlockSpec(wg.shape, lambda i: (0, 0)),
                  pl.BlockSpec(fc2_b.shape, lambda i: (0, 0)),
                  pl.BlockSpec(wo.shape, lambda i: (0, 0)),
                  pl.BlockSpec(out_b.shape, lambda i: (0, 0))],
        out_specs=pl.BlockSpec((tb, 10), lambda i: (i, 0)),
        scratch_shapes=[pltpu.VMEM((tb, 12 * LANES), OP_DT),
                        pltpu.VMEM((tb, 4 * LANES), OP_DT)],
        compiler_params=pltpu.CompilerParams(
            dimension_semantics=("parallel",)),
    )(x3, w1, b1, w2, b2, wf, fc1_b, wg, fc2_b, wo, out_b)


# speedup vs baseline: 1.6436x; 1.0243x over previous
import numpy as np
import jax
import jax.numpy as jnp
from jax.experimental import pallas as pl
from jax.experimental.pallas import tpu as pltpu

LANES = 128
TB = 1024
OP_DT = jnp.bfloat16


def _conv1_tabs():
    e1 = np.zeros((128, 256), np.float32)
    for p in range(2):
        for o in range(6):
            for jp in range(12):
                e1[o, p * 128 + o * 12 + jp] = 1.0
    c1 = np.zeros((5, 140, 25), np.float32)
    m1 = np.zeros((5, 140, 256), np.float32)
    for dj in range(5):
        for di in range(5):
            for jj in range(28):
                c1[dj, di * 28 + jj, di * 5 + dj] = 1.0
        for p in range(2):
            for o in range(6):
                for jp in range(12):
                    col = p * 128 + o * 12 + jp
                    jj = 2 * jp + p + dj
                    for di in range(5):
                        m1[dj, di * 28 + jj, col] = 1.0
    return e1, c1, m1


def _conv2_tabs():
    e2 = np.zeros((128, 256), np.float32)
    for p in range(2):
        for o2 in range(12):
            for j2p in range(4):
                e2[o2, p * 128 + o2 * 4 + j2p] = 1.0
    c2 = np.zeros((5, 640, 150), np.float32)
    m2 = np.zeros((5, 640, 256), np.float32)
    for dj in range(5):
        for di in range(5):
            for c in range(6):
                for ww in range(12):
                    c2[dj, di * 128 + c * 12 + ww, (di * 5 + dj) * 6 + c] = 1.0
        for p in range(2):
            for o2 in range(12):
                for j2p in range(4):
                    col = p * 128 + o2 * 4 + j2p
                    ww = 2 * j2p + p + dj
                    for di in range(5):
                        for c in range(6):
                            m2[dj, di * 128 + c * 12 + ww, col] = 1.0
    return e2, c2, m2


def _fc1_tab():
    pf = np.zeros((512, 192), np.float32)
    for h in range(4):
        for c in range(12):
            for w in range(4):
                pf[h * 128 + c * 4 + w, h * 48 + w * 12 + c] = 1.0
    return pf


_E1, _C1, _M1 = _conv1_tabs()
_E2, _C2, _M2 = _conv2_tabs()
_PF = _fc1_tab()


def _prep_kernel(c1w, c1b, c2w, c2b, fc1w, fc2w, outw, e1, c1t, m1, e2, c2t,
                 m2, pf, w1_o, b1_o, w2_o, b2_o, wf_o, wg_o, wo_o):
    f32 = jnp.float32
    v1 = jnp.dot(c1w[...], e1[...], preferred_element_type=f32)
    w1 = jnp.zeros((140, 256), f32)
    for dj in range(5):
        w1 = w1 + jnp.dot(c1t[dj], v1, preferred_element_type=f32) * m1[dj]
    w1_o[...] = w1.astype(w1_o.dtype)
    b1_o[...] = jnp.dot(c1b[...], e1[...],
                        preferred_element_type=f32)[:, :128]

    v2 = jnp.dot(c2w[...], e2[...], preferred_element_type=f32)
    w2 = jnp.zeros((640, 256), f32)
    for dj in range(5):
        w2 = w2 + jnp.dot(c2t[dj], v2, preferred_element_type=f32) * m2[dj]
    w2_o[...] = w2.astype(w2_o.dtype)
    b2_o[...] = jnp.dot(c2b[...], e2[...],
                        preferred_element_type=f32)[:, :128]

    wf_o[...] = jnp.dot(pf[...], fc1w[...],
                        preferred_element_type=f32).astype(wf_o.dtype)
    wg_o[...] = fc2w[...].astype(wg_o.dtype)
    wo_o[...] = outw[...].astype(wo_o.dtype)


def _full(shape):
    return pl.BlockSpec(shape, lambda: (0,) * len(shape))


def _prep(c1_w, c1_b, c2_w, c2_b, fc1_w, fc2_w, out_w):
    outs = pl.pallas_call(
        _prep_kernel,
        out_shape=[jax.ShapeDtypeStruct((140, 256), OP_DT),
                   jax.ShapeDtypeStruct((1, 128), jnp.float32),
                   jax.ShapeDtypeStruct((640, 256), OP_DT),
                   jax.ShapeDtypeStruct((1, 128), jnp.float32),
                   jax.ShapeDtypeStruct((512, 128), OP_DT),
                   jax.ShapeDtypeStruct((128, 128), OP_DT),
                   jax.ShapeDtypeStruct((128, 128), OP_DT)],
        in_specs=[_full((25, 128)), _full((1, 128)),
                  _full((150, 128)), _full((1, 128)),
                  _full((192, 128)), _full((128, 128)), _full((128, 128)),
                  _full((128, 256)), _full((5, 140, 25)), _full((5, 140, 256)),
                  _full((128, 256)), _full((5, 640, 150)), _full((5, 640, 256)),
                  _full((512, 192))],
        out_specs=[_full((140, 256)), _full((1, 128)),
                   _full((640, 256)), _full((1, 128)),
                   _full((512, 128)), _full((128, 128)), _full((128, 128))],
    )(c1_w, c1_b, c2_w, c2_b, fc1_w, fc2_w, out_w,
      _E1, _C1, _M1, _E2, _C2, _M2, _PF)
    return outs


def _lenet_kernel(x_ref, w1_ref, b1_ref, w2_ref, b2_ref, wf_ref, bf_ref,
                  wg_ref, bg_ref, wo_ref, bo_ref, o_ref, s1, s2):
    tb = x_ref.shape[0]
    xb = x_ref[...].astype(OP_DT).reshape(tb, 784)
    w1 = w1_ref[...]
    for r in range(12):
        m = None
        for i in (2 * r, 2 * r + 1):
            d = jnp.dot(xb[:, i * 28:i * 28 + 140], w1,
                        preferred_element_type=jnp.float32)
            mm = jnp.maximum(d[:, :LANES], d[:, LANES:])
            m = mm if m is None else jnp.maximum(m, mm)
        s1[:, r * LANES:(r + 1) * LANES] = (
            jnp.maximum(m + b1_ref[...], 0.0).astype(OP_DT))

    w2 = w2_ref[...]
    for r in range(4):
        m = None
        for i in (2 * r, 2 * r + 1):
            d = jnp.dot(s1[:, i * LANES:i * LANES + 640], w2,
                        preferred_element_type=jnp.float32)
            mm = jnp.maximum(d[:, :LANES], d[:, LANES:])
            m = mm if m is None else jnp.maximum(m, mm)
        s2[:, r * LANES:(r + 1) * LANES] = (
            jnp.maximum(m + b2_ref[...], 0.0).astype(OP_DT))

    h = jnp.dot(s2[...], wf_ref[...], preferred_element_type=jnp.float32)
    h = jnp.maximum(h + bf_ref[...], 0.0).astype(OP_DT)
    h = jnp.dot(h, wg_ref[...], preferred_element_type=jnp.float32)
    h = jnp.maximum(h + bg_ref[...], 0.0).astype(OP_DT)
    o = jnp.dot(h, wo_ref[...], preferred_element_type=jnp.float32)
    o_ref[...] = (o + bo_ref[...])[:, :10]


def kernel(c1_w, c1_b, c2_w, c2_b, fc1_w, fc1_b, fc2_w, fc2_b, out_w, out_b, x):
    B = x.shape[0]
    tb = TB if B % TB == 0 else B
    x3 = x.reshape(B, 28, 28)

    w1, b1, w2, b2, wf, wg, wo = _prep(c1_w, c1_b, c2_w, c2_b,
                                       fc1_w, fc2_w, out_w)

    return pl.pallas_call(
        _lenet_kernel,
        out_shape=jax.ShapeDtypeStruct((B, 10), jnp.float32),
        grid=(B // tb,),
        in_specs=[pl.BlockSpec((tb, 28, 28), lambda i: (i, 0, 0)),
                  pl.BlockSpec(w1.shape, lambda i: (0, 0)),
                  pl.BlockSpec(b1.shape, lambda i: (0, 0)),
                  pl.BlockSpec(w2.shape, lambda i: (0, 0)),
                  pl.BlockSpec(b2.shape, lambda i: (0, 0)),
                  pl.BlockSpec(wf.shape, lambda i: (0, 0)),
                  pl.BlockSpec(fc1_b.shape, lambda i: (0, 0)),
                  pl.BlockSpec(wg.shape, lambda i: (0, 0)),
                  pl.BlockSpec(fc2_b.shape, lambda i: (0, 0)),
                  pl.BlockSpec(wo.shape, lambda i: (0, 0)),
                  pl.BlockSpec(out_b.shape, lambda i: (0, 0))],
        out_specs=pl.BlockSpec((tb, 10), lambda i: (i, 0)),
        scratch_shapes=[pltpu.VMEM((tb, 12 * LANES), OP_DT),
                        pltpu.VMEM((tb, 4 * LANES), OP_DT)],
        compiler_params=pltpu.CompilerParams(
            dimension_semantics=("parallel",)),
    )(x3, w1, b1, w2, b2, wf, fc1_b, wg, fc2_b, wo, out_b)


# speedup vs baseline: 1.6459x; 1.0014x over previous
import numpy as np
import jax
import jax.numpy as jnp
from jax.experimental import pallas as pl
from jax.experimental.pallas import tpu as pltpu

LANES = 128
TB = 1024
OP_DT = jnp.bfloat16


def _conv1_tabs():
    e1 = np.zeros((128, 256), np.float32)
    for p in range(2):
        for o in range(6):
            for jp in range(12):
                e1[o, p * 128 + o * 12 + jp] = 1.0
    c1 = np.zeros((5, 140, 25), np.float32)
    m1 = np.zeros((5, 140, 256), np.float32)
    for dj in range(5):
        for di in range(5):
            for jj in range(28):
                c1[dj, di * 28 + jj, di * 5 + dj] = 1.0
        for p in range(2):
            for o in range(6):
                for jp in range(12):
                    col = p * 128 + o * 12 + jp
                    jj = 2 * jp + p + dj
                    for di in range(5):
                        m1[dj, di * 28 + jj, col] = 1.0
    return e1, c1, m1


def _conv2_tabs():
    e2 = np.zeros((128, 256), np.float32)
    for p in range(2):
        for o2 in range(12):
            for j2p in range(4):
                e2[o2, p * 128 + o2 * 4 + j2p] = 1.0
    c2 = np.zeros((5, 640, 150), np.float32)
    m2 = np.zeros((5, 640, 256), np.float32)
    for dj in range(5):
        for di in range(5):
            for c in range(6):
                for ww in range(12):
                    c2[dj, di * 128 + c * 12 + ww, (di * 5 + dj) * 6 + c] = 1.0
        for p in range(2):
            for o2 in range(12):
                for j2p in range(4):
                    col = p * 128 + o2 * 4 + j2p
                    ww = 2 * j2p + p + dj
                    for di in range(5):
                        for c in range(6):
                            m2[dj, di * 128 + c * 12 + ww, col] = 1.0
    return e2, c2, m2


def _fc1_tab():
    pf = np.zeros((512, 192), np.float32)
    for h in range(4):
        for c in range(12):
            for w in range(4):
                pf[h * 128 + c * 4 + w, h * 48 + w * 12 + c] = 1.0
    return pf


_E1, _C1, _M1 = _conv1_tabs()
_E2, _C2, _M2 = _conv2_tabs()
_PF = _fc1_tab()


def _prep_kernel(c1w, c1b, c2w, c2b, fc1w, fc2w, outw, e1, c1t, m1, e2, c2t,
                 m2, pf, w1_o, b1_o, w2_o, b2_o, wf_o, wg_o, wo_o):
    f32 = jnp.float32
    v1 = jnp.dot(c1w[...], e1[...], preferred_element_type=f32)
    w1 = jnp.zeros((140, 256), f32)
    for dj in range(5):
        w1 = w1 + jnp.dot(c1t[dj], v1, preferred_element_type=f32) * m1[dj]
    w1_o[...] = w1.astype(w1_o.dtype)
    b1_o[...] = jnp.dot(c1b[...], e1[...],
                        preferred_element_type=f32)[:, :128]

    v2 = jnp.dot(c2w[...], e2[...], preferred_element_type=f32)
    w2 = jnp.zeros((640, 256), f32)
    for dj in range(5):
        w2 = w2 + jnp.dot(c2t[dj], v2, preferred_element_type=f32) * m2[dj]
    w2_o[...] = w2.astype(w2_o.dtype)
    b2_o[...] = jnp.dot(c2b[...], e2[...],
                        preferred_element_type=f32)[:, :128]

    wf_o[...] = jnp.dot(pf[...], fc1w[...],
                        preferred_element_type=f32).astype(wf_o.dtype)
    wg_o[...] = fc2w[...].astype(wg_o.dtype)
    wo_o[...] = outw[...].astype(wo_o.dtype)


def _full(shape):
    return pl.BlockSpec(shape, lambda: (0,) * len(shape))


def _prep(c1_w, c1_b, c2_w, c2_b, fc1_w, fc2_w, out_w):
    outs = pl.pallas_call(
        _prep_kernel,
        out_shape=[jax.ShapeDtypeStruct((140, 256), OP_DT),
                   jax.ShapeDtypeStruct((1, 128), jnp.float32),
                   jax.ShapeDtypeStruct((640, 256), OP_DT),
                   jax.ShapeDtypeStruct((1, 128), jnp.float32),
                   jax.ShapeDtypeStruct((512, 128), OP_DT),
                   jax.ShapeDtypeStruct((128, 128), OP_DT),
                   jax.ShapeDtypeStruct((128, 128), OP_DT)],
        in_specs=[_full((25, 128)), _full((1, 128)),
                  _full((150, 128)), _full((1, 128)),
                  _full((192, 128)), _full((128, 128)), _full((128, 128)),
                  _full((128, 256)), _full((5, 140, 25)), _full((5, 140, 256)),
                  _full((128, 256)), _full((5, 640, 150)), _full((5, 640, 256)),
                  _full((512, 192))],
        out_specs=[_full((140, 256)), _full((1, 128)),
                   _full((640, 256)), _full((1, 128)),
                   _full((512, 128)), _full((128, 128)), _full((128, 128))],
    )(c1_w, c1_b, c2_w, c2_b, fc1_w, fc2_w, out_w,
      _E1, _C1, _M1, _E2, _C2, _M2, _PF)
    return outs


def _lenet_kernel(x_ref, w1_ref, b1_ref, w2_ref, b2_ref, wf_ref, bf_ref,
                  wg_ref, bg_ref, wo_ref, bo_ref, o_ref, s1, s2):
    tb = x_ref.shape[0]
    xb = x_ref[...].astype(OP_DT).reshape(tb, 784)
    w1 = w1_ref[...]
    for r in range(12):
        m = None
        for i in (2 * r, 2 * r + 1):
            d = jnp.dot(xb[:, i * 28:i * 28 + 140], w1,
                        preferred_element_type=jnp.float32)
            mm = jnp.maximum(d[:, :LANES], d[:, LANES:])
            m = mm if m is None else jnp.maximum(m, mm)
        s1[:, r * LANES:(r + 1) * LANES] = (
            jnp.maximum(m + b1_ref[...], 0.0).astype(OP_DT))

    w2 = w2_ref[...]
    for r in range(4):
        m = None
        for i in (2 * r, 2 * r + 1):
            d = jnp.dot(s1[:, i * LANES:i * LANES + 640], w2,
                        preferred_element_type=jnp.float32)
            mm = jnp.maximum(d[:, :LANES], d[:, LANES:])
            m = mm if m is None else jnp.maximum(m, mm)
        s2[:, r * LANES:(r + 1) * LANES] = (
            jnp.maximum(m + b2_ref[...], 0.0).astype(OP_DT))

    h = jnp.dot(s2[...], wf_ref[...], preferred_element_type=jnp.float32)
    h = jnp.maximum(h + bf_ref[...], 0.0).astype(OP_DT)
    h = jnp.dot(h, wg_ref[...], preferred_element_type=jnp.float32)
    h = jnp.maximum(h + bg_ref[...], 0.0).astype(OP_DT)
    o = jnp.dot(h, wo_ref[...], preferred_element_type=jnp.float32)
    o_ref[...] = (o + bo_ref[...])[:, :10]


def kernel(c1_w, c1_b, c2_w, c2_b, fc1_w, fc1_b, fc2_w, fc2_b, out_w, out_b, x):
    B = x.shape[0]
    tb = TB if B % TB == 0 else B
    x3 = x.transpose(1, 0, 2, 3).reshape(B, 28, 28)

    w1, b1, w2, b2, wf, wg, wo = _prep(c1_w, c1_b, c2_w, c2_b,
                                       fc1_w, fc2_w, out_w)

    return pl.pallas_call(
        _lenet_kernel,
        out_shape=jax.ShapeDtypeStruct((B, 10), jnp.float32),
        grid=(B // tb,),
        in_specs=[pl.BlockSpec((tb, 28, 28), lambda i: (i, 0, 0)),
                  pl.BlockSpec(w1.shape, lambda i: (0, 0)),
                  pl.BlockSpec(b1.shape, lambda i: (0, 0)),
                  pl.BlockSpec(w2.shape, lambda i: (0, 0)),
                  pl.BlockSpec(b2.shape, lambda i: (0, 0)),
                  pl.BlockSpec(wf.shape, lambda i: (0, 0)),
                  pl.BlockSpec(fc1_b.shape, lambda i: (0, 0)),
                  pl.BlockSpec(wg.shape, lambda i: (0, 0)),
                  pl.BlockSpec(fc2_b.shape, lambda i: (0, 0)),
                  pl.BlockSpec(wo.shape, lambda i: (0, 0)),
                  pl.BlockSpec(out_b.shape, lambda i: (0, 0))],
        out_specs=pl.BlockSpec((tb, 10), lambda i: (i, 0)),
        scratch_shapes=[pltpu.VMEM((tb, 12 * LANES), OP_DT),
                        pltpu.VMEM((tb, 4 * LANES), OP_DT)],
        compiler_params=pltpu.CompilerParams(
            dimension_semantics=("parallel",)),
    )(x3, w1, b1, w2, b2, wf, fc1_b, wg, fc2_b, wo, out_b)
